# baseline (speedup 1.0000x reference)
import dataclasses
import functools

import jax
import jax.numpy as jnp
import numpy as np
from jax import lax
from jax.experimental import pallas as pl
from jax.experimental.pallas import tpu as pltpu
from jax.experimental.pallas import tpu_sc as plsc

F32 = jnp.float32
I32 = jnp.int32


def _sc_compiler_params():
    cp = pltpu.CompilerParams()
    if "needs_layout_passes" in pltpu.CompilerParams.__dataclass_fields__:
        cp = dataclasses.replace(cp, needs_layout_passes=False)
    return cp


def _k1_body(x_ref, w_ref, b_ref, wa_ref, wp_ref, lab_ref, wbar_ref):
    ip = x_ref[0]
    cols = [ip[:, dy:dy + 224, dx:dx + 224]
            for dy in range(3) for dx in range(3)]
    xx = jnp.stack(cols, axis=0)
    xx = xx.reshape(27, 224 * 224)
    logits = jnp.dot(w_ref[...], xx, preferred_element_type=F32) + b_ref[...]
    lab_ref[0, 0, :] = jnp.argmax(logits, axis=0).astype(I32)

    @pl.when(pl.program_id(0) == 0)
    def _():
        wbar_ref[...] = jnp.dot(wa_ref[...], wp_ref[...],
                                preferred_element_type=F32)


def _run_k1(imgp, W27, b64, wa, Wp2, B, HW):
    return pl.pallas_call(
        _k1_body,
        grid=(B,),
        in_specs=[
            pl.BlockSpec((1, 3, 226, 226), lambda b: (b, 0, 0, 0)),
            pl.BlockSpec((64, 27), lambda b: (0, 0)),
            pl.BlockSpec((64, 1), lambda b: (0, 0)),
            pl.BlockSpec((1, 96), lambda b: (0, 0)),
            pl.BlockSpec((96, 768), lambda b: (0, 0)),
        ],
        out_specs=[
            pl.BlockSpec((1, 1, HW), lambda b: (b, 0, 0)),
            pl.BlockSpec((1, 768), lambda b: (0, 0)),
        ],
        out_shape=[
            jax.ShapeDtypeStruct((B, 1, HW), I32),
            jax.ShapeDtypeStruct((1, 768), F32),
        ],
    )(imgp, W27, b64, wa, Wp2)


def _run_ksc(labels, pid, off, img_flat, wbar, zeros256, zeros768,
             B, HW, TPI, NPIX):
    mesh = plsc.VectorSubcoreMesh(core_axis_name="c", subcore_axis_name="s")
    SB = 64 * 256
    CH = SB // TPI

    @functools.partial(
        pl.kernel,
        mesh=mesh,
        compiler_params=_sc_compiler_params(),
        out_type=[
            jax.ShapeDtypeStruct((B, TPI, 64, 768), F32),
            jax.ShapeDtypeStruct((B * TPI * 64,), F32),
            jax.ShapeDtypeStruct((B * TPI * SB,), F32),
        ],
        scratch_types=[
            pltpu.VMEM((NPIX,), I32),
            pltpu.VMEM((NPIX,), I32),
            pltpu.VMEM((NPIX,), I32),
            pltpu.VMEM((NPIX,), F32),
            pltpu.VMEM((NPIX,), F32),
            pltpu.VMEM((NPIX,), F32),
            pltpu.VMEM((768,), F32),
            pltpu.VMEM((SB,), F32),
            pltpu.VMEM((CH,), F32),
            pltpu.VMEM((64, 768), F32),
            pltpu.VMEM((64,), F32),
            pltpu.VMEM_SHARED((2 * SB,), F32),
            pltpu.SemaphoreType.DMA,
        ],
    )
    def ksc(lab_hbm, pid_hbm, off_hbm, img_hbm, wbar_hbm, z256_hbm, z768_hbm,
            zw_out, c_out, ws_out,
            lbl_v, pid_v, off_v, i0_v, i1_v, i2_v, wbar_v, sacc, chk_v,
            zacc, cacc, shared, sem):
        sid = lax.axis_index("s")
        wid = lax.axis_index("c") * 16 + sid
        b = wid // TPI
        part = wid % TPI
        slot = sid // TPI
        base = part * NPIX
        cps = [
            pltpu.async_copy(lab_hbm.at[pl.ds(b * HW + base, NPIX)], lbl_v,
                             sem),
            pltpu.async_copy(pid_hbm.at[pl.ds(base, NPIX)], pid_v, sem),
            pltpu.async_copy(off_hbm.at[pl.ds(base, NPIX)], off_v, sem),
            pltpu.async_copy(img_hbm.at[pl.ds((b * 3 + 0) * HW + base, NPIX)],
                             i0_v, sem),
            pltpu.async_copy(img_hbm.at[pl.ds((b * 3 + 1) * HW + base, NPIX)],
                             i1_v, sem),
            pltpu.async_copy(img_hbm.at[pl.ds((b * 3 + 2) * HW + base, NPIX)],
                             i2_v, sem),
            pltpu.async_copy(wbar_hbm, wbar_v, sem),
            pltpu.async_copy(z256_hbm, sacc, sem),
            pltpu.async_copy(z768_hbm, zacc, sem),
        ]
        for cp in cps:
            cp.wait()

        @pl.loop(0, 64, step=16)
        def _(j):
            cacc[pl.ds(j, 16)] = jnp.zeros((16,), F32)

        ones = jnp.full((16,), 1.0, F32)
        lane16 = lax.broadcasted_iota(I32, (16,), 0)
        zero16 = jnp.zeros((16,), F32)

        @plsc.parallel_loop(0, NPIX, 16, unroll=8)
        def _(i):
            sl = pl.ds(i, 16)
            lbl = lbl_v[sl]
            pidv = pid_v[sl]
            offv = off_v[sl]
            w0 = plsc.load_gather(wbar_v, [offv])
            w1 = plsc.load_gather(wbar_v, [offv + 256])
            w2 = plsc.load_gather(wbar_v, [offv + 512])
            u = i0_v[sl] * w0 + i1_v[sl] * w1 + i2_v[sl] * w2
            plsc.addupdate_scatter(sacc, [lbl * 256 + pidv], u)
            plsc.store_scatter(cacc, [lbl], ones)

        pltpu.sync_copy(sacc, ws_out.at[pl.ds(wid * SB, SB)])
        plsc.subcore_barrier()
        pltpu.sync_copy(ws_out.at[pl.ds((b * TPI) * SB + part * CH, CH)],
                        sacc.at[pl.ds(part * CH, CH)])
        for p in range(1, TPI):
            pltpu.sync_copy(
                ws_out.at[pl.ds((b * TPI + p) * SB + part * CH, CH)], chk_v)

            @pl.loop(0, CH, step=64)
            def _(j):
                for q in range(4):
                    dst = pl.ds(part * CH + j + q * 16, 16)
                    sacc[dst] = sacc[dst] + chk_v[pl.ds(j + q * 16, 16)]

        pltpu.sync_copy(sacc.at[pl.ds(part * CH, CH)],
                        shared.at[pl.ds(slot * SB + part * CH, CH)])
        plsc.subcore_barrier()
        pltpu.sync_copy(shared.at[pl.ds(slot * SB, SB)], sacc)

        valid12 = lane16 < 4

        @pl.loop(0, 49)
        def _(s):
            r0 = s * 256
            m = jnp.full((16,), -1e30, F32)
            for j in range(12):
                m = jnp.maximum(m, sacc[pl.ds(r0 + j * 16, 16)])
            m = jnp.maximum(
                m, jnp.where(valid12, sacc[pl.ds(r0 + 192, 16)], -1e30))
            mx = jnp.max(m)
            acc = zero16
            for j in range(12):
                e = jnp.exp(sacc[pl.ds(r0 + j * 16, 16)] - mx)
                sacc[pl.ds(r0 + j * 16, 16)] = e
                acc = acc + e
            e12 = jnp.where(valid12,
                            jnp.exp(sacc[pl.ds(r0 + 192, 16)] - mx), 0.0)
            sacc[pl.ds(r0 + 192, 16)] = e12
            acc = acc + e12
            inv = jnp.full((16,), 1.0, F32) / (zero16 + jnp.sum(acc))
            for j in range(13):
                sacc[pl.ds(r0 + j * 16, 16)] = (
                    sacc[pl.ds(r0 + j * 16, 16)] * inv)

        @plsc.parallel_loop(0, NPIX, 16, unroll=8)
        def _(i):
            sl = pl.ds(i, 16)
            lbl = lbl_v[sl]
            pidv = pid_v[sl]
            offv = off_v[sl]
            a = plsc.load_gather(sacc, [lbl * 256 + pidv])
            plsc.addupdate_scatter(zacc, [lbl, offv], i0_v[sl] * a)
            plsc.addupdate_scatter(zacc, [lbl, offv + 256], i1_v[sl] * a)
            plsc.addupdate_scatter(zacc, [lbl, offv + 512], i2_v[sl] * a)

        pltpu.sync_copy(zacc, zw_out.at[b, part])
        pltpu.sync_copy(cacc, c_out.at[pl.ds(wid * 64, 64)])

    return ksc(labels, pid, off, img_flat, wbar, zeros256, zeros768)[:2]


def _k5_body(zw_ref, cnt_ref, wq_ref, bp_ref, wo_ref, bo_ref, out_ref):
    Z = jnp.sum(zw_ref[0], axis=0)
    pooled = jnp.dot(Z, wq_ref[...], preferred_element_type=F32) + bp_ref[...]
    feats = jnp.dot(pooled, wo_ref[...], preferred_element_type=F32) + bo_ref[...]
    cnt = jnp.sum(cnt_ref[0], axis=0)
    present = cnt > 0.5
    pr = present.astype(F32)[None, :]
    r = lax.broadcasted_iota(I32, (64, 64), 0)
    c = lax.broadcasted_iota(I32, (64, 64), 1)
    pos = jnp.sum(jnp.where(c <= r, pr, 0.0), axis=1)
    perm = jnp.where((pos[None, :] - 1.0 == r.astype(F32)) & present[None, :],
                     1.0, 0.0)
    outb = jnp.dot(perm, feats, preferred_element_type=F32)
    out_ref[0] = outb[:49]


def _run_k5(zw_part, cnt_part, Wq, bp, Wo, bo, B, TPI):
    return pl.pallas_call(
        _k5_body,
        grid=(B,),
        in_specs=[
            pl.BlockSpec((1, TPI, 64, 768), lambda b: (b, 0, 0, 0)),
            pl.BlockSpec((1, TPI, 64), lambda b: (b, 0, 0)),
            pl.BlockSpec((768, 96), lambda b: (0, 0)),
            pl.BlockSpec((1, 96), lambda b: (0, 0)),
            pl.BlockSpec((96, 768), lambda b: (0, 0)),
            pl.BlockSpec((1, 768), lambda b: (0, 0)),
        ],
        out_specs=pl.BlockSpec((1, 49, 768), lambda b: (b, 0, 0)),
        out_shape=jax.ShapeDtypeStruct((B, 49, 768), F32),
    )(zw_part, cnt_part, Wq, bp, Wo, bo)


def kernel(img, W_spix, b_spix, W_patch, b_patch, w_attn, W_out, b_out):
    B, C, H, Wd = img.shape
    S = W_spix.shape[0]
    stem = W_patch.shape[0]
    patch = W_patch.shape[2]
    HP = H // patch
    HW = H * Wd
    TPI = 32 // B
    NPIX = HW // TPI

    y = np.arange(H)[:, None]
    x = np.arange(Wd)[None, :]
    pid = jnp.asarray(((y // patch) * HP + (x // patch))
                      .astype(np.int32).reshape(-1))
    off = jnp.asarray(((y % patch) * patch + (x % patch))
                      .astype(np.int32).reshape(-1))

    imgp = jnp.pad(img, ((0, 0), (0, 0), (1, 1), (1, 1)))

    W27 = W_spix.transpose(0, 2, 3, 1).reshape(S, 9 * C)
    W27 = jnp.pad(W27, ((0, 64 - S), (0, 0)))
    b64 = jnp.pad(b_spix, (0, 64 - S),
                  constant_values=-1e30)[:, None]
    Wp2 = W_patch.reshape(stem, C * patch * patch)

    labels, wbar = _run_k1(imgp, W27, b64, w_attn[None, :], Wp2, B, HW)
    labels = labels.reshape(B * HW)
    wbar = wbar.reshape(768)

    img_flat = img.reshape(B * C * HW)
    zeros256 = jnp.zeros((64 * 256,), F32)
    zeros768 = jnp.zeros((64, 768), F32)

    zw_part, cnt_part = _run_ksc(labels, pid, off, img_flat, wbar,
                                 zeros256, zeros768, B, HW, TPI, NPIX)
    out = _run_k5(zw_part, cnt_part.reshape(B, TPI, 64), Wp2.T,
                  b_patch[None, :], W_out, b_out[None, :], B, TPI)
    return out


# speedup vs baseline: 1.0737x; 1.0737x over previous
import dataclasses
import functools

import jax
import jax.numpy as jnp
import numpy as np
from jax import lax
from jax.experimental import pallas as pl
from jax.experimental.pallas import tpu as pltpu
from jax.experimental.pallas import tpu_sc as plsc

F32 = jnp.float32
I32 = jnp.int32


def _sc_compiler_params():
    cp = pltpu.CompilerParams()
    if "needs_layout_passes" in pltpu.CompilerParams.__dataclass_fields__:
        cp = dataclasses.replace(cp, needs_layout_passes=False)
    return cp


def _k1_body(x_ref, w_ref, b_ref, wa_ref, wp_ref, lab_ref, wbar_ref):
    ip = x_ref[0]
    cols = [ip[:, dy:dy + 224, dx:dx + 224]
            for dy in range(3) for dx in range(3)]
    xx = jnp.stack(cols, axis=0)
    xx = xx.reshape(27, 224 * 224)
    logits = jnp.dot(w_ref[...], xx, preferred_element_type=F32) + b_ref[...]
    lab_ref[0, 0, :] = jnp.argmax(logits, axis=0).astype(I32)

    @pl.when(pl.program_id(0) == 0)
    def _():
        wbar_ref[...] = jnp.dot(wa_ref[...], wp_ref[...],
                                preferred_element_type=F32)


def _run_k1(imgp, W27, b64, wa, Wp2, B, HW):
    return pl.pallas_call(
        _k1_body,
        grid=(B,),
        in_specs=[
            pl.BlockSpec((1, 3, 226, 226), lambda b: (b, 0, 0, 0)),
            pl.BlockSpec((64, 27), lambda b: (0, 0)),
            pl.BlockSpec((64, 1), lambda b: (0, 0)),
            pl.BlockSpec((1, 96), lambda b: (0, 0)),
            pl.BlockSpec((96, 768), lambda b: (0, 0)),
        ],
        out_specs=[
            pl.BlockSpec((1, 1, HW), lambda b: (b, 0, 0)),
            pl.BlockSpec((1, 768), lambda b: (0, 0)),
        ],
        out_shape=[
            jax.ShapeDtypeStruct((B, 1, HW), I32),
            jax.ShapeDtypeStruct((1, 768), F32),
        ],
    )(imgp, W27, b64, wa, Wp2)


def _run_ksc(labels, pid, off, img_flat, wbar, zeros256, zeros768,
             B, HW, TPI, NPIX):
    mesh = plsc.VectorSubcoreMesh(core_axis_name="c", subcore_axis_name="s")
    SB = 64 * 256
    CH = SB // TPI

    @functools.partial(
        pl.kernel,
        mesh=mesh,
        compiler_params=_sc_compiler_params(),
        out_type=[
            jax.ShapeDtypeStruct((B, TPI, 64, 768), F32),
            jax.ShapeDtypeStruct((B * TPI * 64,), F32),
            jax.ShapeDtypeStruct((B * TPI * SB,), F32),
        ],
        scratch_types=[
            pltpu.VMEM((NPIX,), I32),
            pltpu.VMEM((NPIX,), I32),
            pltpu.VMEM((NPIX,), I32),
            pltpu.VMEM((NPIX,), F32),
            pltpu.VMEM((NPIX,), F32),
            pltpu.VMEM((NPIX,), F32),
            pltpu.VMEM((768,), F32),
            pltpu.VMEM((SB,), F32),
            pltpu.VMEM(((TPI - 1) * CH,), F32),
            pltpu.VMEM((64, 768), F32),
            pltpu.VMEM((64,), F32),
            pltpu.VMEM_SHARED((2 * SB,), F32),
            pltpu.SemaphoreType.DMA,
        ],
    )
    def ksc(lab_hbm, pid_hbm, off_hbm, img_hbm, wbar_hbm, z256_hbm, z768_hbm,
            zw_out, c_out, ws_out,
            lbl_v, pid_v, off_v, i0_v, i1_v, i2_v, wbar_v, sacc, chk_v,
            zacc, cacc, shared, sem):
        sid = lax.axis_index("s")
        wid = lax.axis_index("c") * 16 + sid
        b = wid // TPI
        part = wid % TPI
        slot = sid // TPI
        base = part * NPIX
        cps = [
            pltpu.async_copy(lab_hbm.at[pl.ds(b * HW + base, NPIX)], lbl_v,
                             sem),
            pltpu.async_copy(pid_hbm.at[pl.ds(base, NPIX)], pid_v, sem),
            pltpu.async_copy(off_hbm.at[pl.ds(base, NPIX)], off_v, sem),
            pltpu.async_copy(img_hbm.at[pl.ds((b * 3 + 0) * HW + base, NPIX)],
                             i0_v, sem),
            pltpu.async_copy(img_hbm.at[pl.ds((b * 3 + 1) * HW + base, NPIX)],
                             i1_v, sem),
            pltpu.async_copy(img_hbm.at[pl.ds((b * 3 + 2) * HW + base, NPIX)],
                             i2_v, sem),
            pltpu.async_copy(wbar_hbm, wbar_v, sem),
            pltpu.async_copy(z256_hbm, sacc, sem),
            pltpu.async_copy(z768_hbm, zacc, sem),
        ]
        for cp in cps:
            cp.wait()

        @pl.loop(0, 64, step=16)
        def _(j):
            cacc[pl.ds(j, 16)] = jnp.zeros((16,), F32)

        ones = jnp.full((16,), 1.0, F32)
        lane16 = lax.broadcasted_iota(I32, (16,), 0)
        zero16 = jnp.zeros((16,), F32)

        @plsc.parallel_loop(0, NPIX, 16, unroll=4)
        def _(i):
            sl = pl.ds(i, 16)
            lbl = lbl_v[sl]
            pidv = pid_v[sl]
            offv = off_v[sl]
            w0 = plsc.load_gather(wbar_v, [offv])
            w1 = plsc.load_gather(wbar_v, [offv + 256])
            w2 = plsc.load_gather(wbar_v, [offv + 512])
            u = i0_v[sl] * w0 + i1_v[sl] * w1 + i2_v[sl] * w2
            plsc.addupdate_scatter(sacc, [lbl * 256 + pidv], u)
            plsc.store_scatter(cacc, [lbl], ones)

        pltpu.sync_copy(sacc, ws_out.at[pl.ds(wid * SB, SB)])
        plsc.subcore_barrier()
        ccps = [
            pltpu.async_copy(
                ws_out.at[pl.ds(
                    (b * TPI + jnp.where(p >= part, p + 1, p)) * SB
                    + part * CH, CH)],
                chk_v.at[pl.ds(p * CH, CH)], sem)
            for p in range(TPI - 1)
        ]
        for cp in ccps:
            cp.wait()

        @pl.loop(0, CH, step=64)
        def _(j):
            for q in range(4):
                dst = pl.ds(part * CH + j + q * 16, 16)
                acc = sacc[dst]
                for p in range(TPI - 1):
                    acc = acc + chk_v[pl.ds(p * CH + j + q * 16, 16)]
                sacc[dst] = acc

        pltpu.sync_copy(sacc.at[pl.ds(part * CH, CH)],
                        shared.at[pl.ds(slot * SB + part * CH, CH)])
        plsc.subcore_barrier()
        pltpu.sync_copy(shared.at[pl.ds(slot * SB, SB)], sacc)

        valid12 = lane16 < 4

        @pl.loop(0, 49)
        def _(s):
            r0 = s * 256
            m = jnp.full((16,), -1e30, F32)
            for j in range(12):
                m = jnp.maximum(m, sacc[pl.ds(r0 + j * 16, 16)])
            m = jnp.maximum(
                m, jnp.where(valid12, sacc[pl.ds(r0 + 192, 16)], -1e30))
            mx = jnp.max(m)
            acc = zero16
            for j in range(12):
                e = jnp.exp(sacc[pl.ds(r0 + j * 16, 16)] - mx)
                sacc[pl.ds(r0 + j * 16, 16)] = e
                acc = acc + e
            e12 = jnp.where(valid12,
                            jnp.exp(sacc[pl.ds(r0 + 192, 16)] - mx), 0.0)
            sacc[pl.ds(r0 + 192, 16)] = e12
            acc = acc + e12
            inv = jnp.full((16,), 1.0, F32) / (zero16 + jnp.sum(acc))
            for j in range(13):
                sacc[pl.ds(r0 + j * 16, 16)] = (
                    sacc[pl.ds(r0 + j * 16, 16)] * inv)

        @plsc.parallel_loop(0, NPIX, 16, unroll=4)
        def _(i):
            sl = pl.ds(i, 16)
            lbl = lbl_v[sl]
            pidv = pid_v[sl]
            offv = off_v[sl]
            a = plsc.load_gather(sacc, [lbl * 256 + pidv])
            plsc.addupdate_scatter(zacc, [lbl, offv], i0_v[sl] * a)
            plsc.addupdate_scatter(zacc, [lbl, offv + 256], i1_v[sl] * a)
            plsc.addupdate_scatter(zacc, [lbl, offv + 512], i2_v[sl] * a)

        pltpu.sync_copy(zacc, zw_out.at[b, part])
        pltpu.sync_copy(cacc, c_out.at[pl.ds(wid * 64, 64)])

    return ksc(labels, pid, off, img_flat, wbar, zeros256, zeros768)[:2]


def _k5_body(zw_ref, cnt_ref, wq_ref, bp_ref, wo_ref, bo_ref, out_ref):
    Z = jnp.sum(zw_ref[0], axis=0)
    pooled = jnp.dot(Z, wq_ref[...], preferred_element_type=F32) + bp_ref[...]
    feats = jnp.dot(pooled, wo_ref[...], preferred_element_type=F32) + bo_ref[...]
    cnt = jnp.sum(cnt_ref[0], axis=0)
    present = cnt > 0.5
    pr = present.astype(F32)[None, :]
    r = lax.broadcasted_iota(I32, (64, 64), 0)
    c = lax.broadcasted_iota(I32, (64, 64), 1)
    pos = jnp.sum(jnp.where(c <= r, pr, 0.0), axis=1)
    perm = jnp.where((pos[None, :] - 1.0 == r.astype(F32)) & present[None, :],
                     1.0, 0.0)
    outb = jnp.dot(perm, feats, preferred_element_type=F32)
    out_ref[0] = outb[:49]


def _run_k5(zw_part, cnt_part, Wq, bp, Wo, bo, B, TPI):
    return pl.pallas_call(
        _k5_body,
        grid=(B,),
        in_specs=[
            pl.BlockSpec((1, TPI, 64, 768), lambda b: (b, 0, 0, 0)),
            pl.BlockSpec((1, TPI, 64), lambda b: (b, 0, 0)),
            pl.BlockSpec((768, 96), lambda b: (0, 0)),
            pl.BlockSpec((1, 96), lambda b: (0, 0)),
            pl.BlockSpec((96, 768), lambda b: (0, 0)),
            pl.BlockSpec((1, 768), lambda b: (0, 0)),
        ],
        out_specs=pl.BlockSpec((1, 49, 768), lambda b: (b, 0, 0)),
        out_shape=jax.ShapeDtypeStruct((B, 49, 768), F32),
    )(zw_part, cnt_part, Wq, bp, Wo, bo)


def kernel(img, W_spix, b_spix, W_patch, b_patch, w_attn, W_out, b_out):
    B, C, H, Wd = img.shape
    S = W_spix.shape[0]
    stem = W_patch.shape[0]
    patch = W_patch.shape[2]
    HP = H // patch
    HW = H * Wd
    TPI = 32 // B
    NPIX = HW // TPI

    y = np.arange(H)[:, None]
    x = np.arange(Wd)[None, :]
    pid = jnp.asarray(((y // patch) * HP + (x // patch))
                      .astype(np.int32).reshape(-1))
    off = jnp.asarray(((y % patch) * patch + (x % patch))
                      .astype(np.int32).reshape(-1))

    imgp = jnp.pad(img, ((0, 0), (0, 0), (1, 1), (1, 1)))

    W27 = W_spix.transpose(0, 2, 3, 1).reshape(S, 9 * C)
    W27 = jnp.pad(W27, ((0, 64 - S), (0, 0)))
    b64 = jnp.pad(b_spix, (0, 64 - S),
                  constant_values=-1e30)[:, None]
    Wp2 = W_patch.reshape(stem, C * patch * patch)

    labels, wbar = _run_k1(imgp, W27, b64, w_attn[None, :], Wp2, B, HW)
    labels = labels.reshape(B * HW)
    wbar = wbar.reshape(768)

    img_flat = img.reshape(B * C * HW)
    zeros256 = jnp.zeros((64 * 256,), F32)
    zeros768 = jnp.zeros((64, 768), F32)

    zw_part, cnt_part = _run_ksc(labels, pid, off, img_flat, wbar,
                                 zeros256, zeros768, B, HW, TPI, NPIX)
    out = _run_k5(zw_part, cnt_part.reshape(B, TPI, 64), Wp2.T,
                  b_patch[None, :], W_out, b_out[None, :], B, TPI)
    return out
